# Initial kernel scaffold; baseline (speedup 1.0000x reference)
import jax
import jax.numpy as jnp
from jax.experimental import pallas as pl


def kernel(x, adj, W0, b0, g0, be0, W1, b1, g1, be1, W2, b2, g2, be2):
    raise NotImplementedError("write your pallas kernel here")


# speedup vs baseline: 1.2119x; 1.2119x over previous
import jax
import jax.numpy as jnp
from jax.experimental import pallas as pl

_BI = 256


def _deg_body(adj_ref, dis_ref):
    s = jnp.sum(adj_ref[:], axis=-1) + 1.0
    deg = jnp.maximum(s, 1.0)
    dis_ref[:] = jax.lax.rsqrt(deg)[:, :, None]


def _prep0_body(x_ref, w_ref, dis_ref, out_ref):
    B, N, Cin = x_ref.shape
    C = w_ref.shape[1]
    xx = x_ref[:].reshape(B * N, Cin)
    h = jnp.dot(xx, w_ref[:], preferred_element_type=jnp.float32)
    out_ref[:] = (h * dis_ref[:].reshape(B * N, 1)).reshape(B, N, C)


def _mm_body(adj_ref, v_ref, dis_ref, bias_ref, out_ref):
    i = pl.program_id(1)
    bi = adj_ref.shape[1]
    a = adj_ref[0]
    v = v_ref[0]
    acc = jnp.dot(a, v, preferred_element_type=jnp.float32)
    self_term = v_ref[0, pl.ds(i * bi, bi), :]
    di = dis_ref[0]
    out_ref[0] = (acc + self_term) * di + bias_ref[:]


def _bn_prep_body(t_ref, g_ref, be_ref, w_ref, dis_ref, out_ref):
    B, N, C = t_ref.shape
    C2 = w_ref.shape[1]
    t = t_ref[:].reshape(B * N, C)
    mean = jnp.mean(t, axis=0, keepdims=True)
    cen = t - mean
    var = jnp.mean(cen * cen, axis=0, keepdims=True)
    xn = cen * jax.lax.rsqrt(var + 1e-5)
    y = jnp.maximum(xn * g_ref[:] + be_ref[:], 0.0)
    vn = jnp.dot(y, w_ref[:], preferred_element_type=jnp.float32)
    out_ref[:] = (vn * dis_ref[:].reshape(B * N, 1)).reshape(B, N, C2)


def _final_body(t_ref, g_ref, be_ref, out_ref):
    B, N, C = t_ref.shape
    t = t_ref[:].reshape(B * N, C)
    mean = jnp.mean(t, axis=0, keepdims=True)
    cen = t - mean
    var = jnp.mean(cen * cen, axis=0, keepdims=True)
    xn = cen * jax.lax.rsqrt(var + 1e-5)
    y = jnp.maximum(xn * g_ref[:] + be_ref[:], 0.0)
    out_ref[:] = jax.nn.sigmoid(y).reshape(B, N, C)


def _layer_matmul(adj, v, dis, bias):
    B, N, _ = adj.shape
    C = v.shape[-1]
    nb = N // _BI
    return pl.pallas_call(
        _mm_body,
        grid=(B, nb),
        in_specs=[
            pl.BlockSpec((1, _BI, N), lambda b, i: (b, i, 0)),
            pl.BlockSpec((1, N, C), lambda b, i: (b, 0, 0)),
            pl.BlockSpec((1, _BI, 1), lambda b, i: (b, i, 0)),
            pl.BlockSpec((1, C), lambda b, i: (0, 0)),
        ],
        out_specs=pl.BlockSpec((1, _BI, C), lambda b, i: (b, i, 0)),
        out_shape=jax.ShapeDtypeStruct((B, N, C), jnp.float32),
    )(adj, v, dis, bias)


def kernel(x, adj, W0, b0, g0, be0, W1, b1, g1, be1, W2, b2, g2, be2):
    B, N, _ = adj.shape
    nb = N // _BI

    dis = pl.pallas_call(
        _deg_body,
        grid=(B, nb),
        in_specs=[pl.BlockSpec((1, _BI, N), lambda b, i: (b, i, 0))],
        out_specs=pl.BlockSpec((1, _BI, 1), lambda b, i: (b, i, 0)),
        out_shape=jax.ShapeDtypeStruct((B, N, 1), jnp.float32),
    )(adj)

    v = pl.pallas_call(
        _prep0_body,
        out_shape=jax.ShapeDtypeStruct((B, N, W0.shape[1]), jnp.float32),
    )(x, W0, dis)

    for (bias, g, be, Wn) in ((b0, g0, be0, W1), (b1, g1, be1, W2)):
        t = _layer_matmul(adj, v, dis, bias.reshape(1, -1))
        v = pl.pallas_call(
            _bn_prep_body,
            out_shape=jax.ShapeDtypeStruct((B, N, Wn.shape[1]), jnp.float32),
        )(t, g.reshape(1, -1), be.reshape(1, -1), Wn, dis)

    t = _layer_matmul(adj, v, dis, b2.reshape(1, -1))
    out = pl.pallas_call(
        _final_body,
        out_shape=jax.ShapeDtypeStruct(t.shape, jnp.float32),
    )(t, g2.reshape(1, -1), be2.reshape(1, -1))
    return out
